# baseline (speedup 1.0000x reference)
import functools

import jax
import jax.numpy as jnp
from jax import lax
from jax.experimental import pallas as pl
from jax.experimental.pallas import tpu as pltpu
from jax.experimental.pallas import tpu_sc as plsc

NUM_ENTITIES = 1000000
EMBED_DIM = 64
BATCH = 16384

_info = plsc.get_sparse_core_info()
_NC, _NS = _info.num_cores, _info.num_subcores
_NW = _NC * _NS
_B_PER_W = BATCH // _NW

_mesh = plsc.VectorSubcoreMesh(core_axis_name="c", subcore_axis_name="s")


@functools.partial(
    pl.kernel,
    mesh=_mesh,
    out_type=jax.ShapeDtypeStruct((BATCH, EMBED_DIM), jnp.float32),
    scratch_types=[
        pltpu.VMEM((_B_PER_W,), jnp.int32),
        pltpu.VMEM((_B_PER_W, EMBED_DIM), jnp.float32),
        pltpu.SemaphoreType.DMA,
    ],
    compiler_params=pltpu.CompilerParams(use_tc_tiling_on_sc=False),
)
def _gather_kernel(idx_hbm, table_hbm, out_hbm, idx_v, buf, sem):
    wid = lax.axis_index("s") * _NC + lax.axis_index("c")
    base = wid * _B_PER_W
    pltpu.sync_copy(idx_hbm.at[pl.ds(base, _B_PER_W)], idx_v)

    def group_body(g, carry):
        v = idx_v[pl.ds(g * 16, 16)]
        t_vec = lax.shift_right_logical(v, 3)
        r_vec = lax.bitwise_and(v, 7)
        for l in range(16):
            pltpu.make_async_copy(
                table_hbm.at[t_vec[l], r_vec[l]], buf.at[g * 16 + l], sem
            ).start()
        return carry

    lax.fori_loop(0, _B_PER_W // 16, group_body, 0)
    pltpu.make_async_copy(out_hbm.at[pl.ds(base, _B_PER_W)], buf, sem).wait()
    pltpu.sync_copy(buf, out_hbm.at[pl.ds(base, _B_PER_W)])


def kernel(entities, entity_embeddings):
    table3 = entity_embeddings.reshape(NUM_ENTITIES // 8, 8, EMBED_DIM)
    return _gather_kernel(entities, table3)


# speedup vs baseline: 2.5695x; 2.5695x over previous
import functools

import jax
import jax.numpy as jnp
from jax import lax
from jax.experimental import pallas as pl
from jax.experimental.pallas import tpu as pltpu
from jax.experimental.pallas import tpu_sc as plsc

NUM_ENTITIES = 1000000
EMBED_DIM = 64
BATCH = 16384

_info = plsc.get_sparse_core_info()
_NC, _NS = _info.num_cores, _info.num_subcores
_NW = _NC * _NS
_B_PER_W = BATCH // _NW

_mesh = plsc.VectorSubcoreMesh(core_axis_name="c", subcore_axis_name="s")


@functools.partial(
    pl.kernel,
    mesh=_mesh,
    out_type=jax.ShapeDtypeStruct((BATCH, EMBED_DIM), jnp.float32),
    scratch_types=[
        pltpu.VMEM((_B_PER_W,), jnp.int32),
        pltpu.VMEM((_B_PER_W, EMBED_DIM), jnp.float32),
        pltpu.SemaphoreType.DMA,
    ],
)
def _gather_kernel(idx_hbm, table_hbm, out_hbm, idx_v, buf, sem):
    wid = lax.axis_index("s") * _NC + lax.axis_index("c")
    base = wid * _B_PER_W
    pltpu.sync_copy(idx_hbm.at[pl.ds(base, _B_PER_W)], idx_v)

    def group_body(g, carry):
        v = idx_v[pl.ds(g * 16, 16)]
        t_vec = lax.shift_right_logical(v, 3)
        r_vec = lax.bitwise_and(v, 7)
        for l in range(16):
            pltpu.make_async_copy(
                table_hbm.at[t_vec[l], r_vec[l]], buf.at[g * 16 + l], sem
            ).start()
        return carry

    lax.fori_loop(0, _B_PER_W // 16, group_body, 0)
    pltpu.make_async_copy(out_hbm.at[pl.ds(base, _B_PER_W)], buf, sem).wait()
    pltpu.sync_copy(buf, out_hbm.at[pl.ds(base, _B_PER_W)])


def kernel(entities, entity_embeddings):
    table3 = entity_embeddings.reshape(NUM_ENTITIES // 8, 8, EMBED_DIM)
    return _gather_kernel(entities, table3)


# speedup vs baseline: 2.5724x; 1.0011x over previous
import functools

import jax
import jax.numpy as jnp
from jax import lax
from jax.experimental import pallas as pl
from jax.experimental.pallas import tpu as pltpu
from jax.experimental.pallas import tpu_sc as plsc

NUM_ENTITIES = 1000000
EMBED_DIM = 64
BATCH = 16384

_info = plsc.get_sparse_core_info()
_NC, _NS = _info.num_cores, _info.num_subcores
_NW = _NC * _NS
_B_PER_W = BATCH // _NW

_mesh = plsc.VectorSubcoreMesh(core_axis_name="c", subcore_axis_name="s")


@functools.partial(
    pl.kernel,
    mesh=_mesh,
    out_type=jax.ShapeDtypeStruct((BATCH, EMBED_DIM), jnp.float32),
    scratch_types=[
        pltpu.VMEM((_B_PER_W,), jnp.int32),
        pltpu.VMEM((_B_PER_W, EMBED_DIM), jnp.float32),
        pltpu.SemaphoreType.DMA,
    ],
)
def _gather_kernel(idx_hbm, table_hbm, out_hbm, idx_v, buf, sem):
    wid = lax.axis_index("s") * _NC + lax.axis_index("c")
    base = wid * _B_PER_W
    pltpu.sync_copy(idx_hbm.at[pl.ds(base, _B_PER_W)], idx_v)

    def group_body(g, carry):
        v = idx_v[pl.ds(g * 16, 16)]
        t_vec = lax.shift_right_logical(v, 4)
        r_vec = lax.bitwise_and(v, 15)
        for l in range(16):
            pltpu.make_async_copy(
                table_hbm.at[t_vec[l], r_vec[l]], buf.at[g * 16 + l], sem
            ).start()
        return carry

    lax.fori_loop(0, _B_PER_W // 16, group_body, 0)
    pltpu.make_async_copy(out_hbm.at[pl.ds(base, _B_PER_W)], buf, sem).wait()
    pltpu.sync_copy(buf, out_hbm.at[pl.ds(base, _B_PER_W)])


def kernel(entities, entity_embeddings):
    table3 = entity_embeddings.reshape(NUM_ENTITIES // 16, 16, EMBED_DIM)
    return _gather_kernel(entities, table3)
